# baseline (device time: 34158 ns/iter reference)
import jax
import jax.numpy as jnp
from jax import lax
from jax.experimental import pallas as pl
from jax.experimental.pallas import tpu as pltpu

N_DEV = 8
M = 768
N = 768

MASKS = (4, 3, 1)
G_ROWS = 256
G_RECV = 224
RS_RECV_OFF = (0, 128, 192)


def _vbit(my, m):
    if m == 4:
        return (my >> 2) & 1
    if m == 3:
        return (my >> 1) & 1
    return (my ^ (my >> 1)) & 1


def kernel(A, B):
    def body(a_ref, b_ref, out_ref, recv_ref, send_sems, recv_sems):
        my = lax.axis_index("i")

        barrier = pltpu.get_barrier_semaphore()
        for m in MASKS:
            pl.semaphore_signal(
                barrier, inc=1,
                device_id=(my ^ m,), device_id_type=pl.DeviceIdType.MESH,
            )
        pl.semaphore_wait(barrier, 3)

        def make_rs(g, s, base):
            half = 128 >> s
            m = MASKS[(s + g) % 3]
            bit = _vbit(my, m)
            keep_off = base + bit * half
            send_off = base + (1 - bit) * half
            recv_off = g * G_RECV + RS_RECV_OFF[s]
            rdma = pltpu.make_async_remote_copy(
                src_ref=out_ref.at[pl.ds(send_off, half), :],
                dst_ref=recv_ref.at[pl.ds(recv_off, half), :],
                send_sem=send_sems.at[g * 6 + s],
                recv_sem=recv_sems.at[g * 6 + s],
                device_id=(my ^ m,),
                device_id_type=pl.DeviceIdType.MESH,
            )
            return rdma, keep_off, recv_off, half

        def make_ag(g, s, base):
            size = 32 << s
            m = MASKS[(2 - s + g) % 3]
            bit = _vbit(my, m)
            rdma = pltpu.make_async_remote_copy(
                src_ref=out_ref.at[pl.ds(base, size), :],
                dst_ref=out_ref.at[pl.ds(base, size), :],
                send_sem=send_sems.at[g * 6 + 3 + s],
                recv_sem=recv_sems.at[g * 6 + 3 + s],
                device_id=(my ^ m,),
                device_id_type=pl.DeviceIdType.MESH,
            )
            return rdma, base - bit * size

        rs = {}
        for g in range(3):
            rows = pl.ds(g * G_ROWS, G_ROWS)
            out_ref[rows, :] = jnp.dot(
                a_ref[rows, :], b_ref[:, :],
                preferred_element_type=jnp.float32,
            )
            rdma, keep, roff, half = make_rs(g, 0, jnp.int32(g * G_ROWS))
            rdma.start()
            rs[g] = (rdma, keep, roff, half)

        for s in (1, 2):
            for g in range(3):
                rdma, keep, roff, half = rs[g]
                rdma.wait()
                out_ref[pl.ds(keep, half), :] = (
                    out_ref[pl.ds(keep, half), :]
                    + recv_ref[pl.ds(roff, half), :]
                )
                nxt, keep2, roff2, half2 = make_rs(g, s, keep)
                nxt.start()
                rs[g] = (nxt, keep2, roff2, half2)

        ag = {}
        for g in range(3):
            rdma, keep, roff, half = rs[g]
            rdma.wait()
            out_ref[pl.ds(keep, half), :] = (
                out_ref[pl.ds(keep, half), :]
                + recv_ref[pl.ds(roff, half), :]
            )
            nxt, base = make_ag(g, 0, keep)
            nxt.start()
            ag[g] = (nxt, base)

        for s in (1, 2):
            for g in range(3):
                rdma, base = ag[g]
                rdma.wait()
                nxt, base2 = make_ag(g, s, base)
                nxt.start()
                ag[g] = (nxt, base2)
        for g in range(3):
            ag[g][0].wait()

    return pl.pallas_call(
        body,
        out_shape=jax.ShapeDtypeStruct((M, N), jnp.float32),
        in_specs=[
            pl.BlockSpec(memory_space=pltpu.VMEM),
            pl.BlockSpec(memory_space=pltpu.VMEM),
        ],
        out_specs=pl.BlockSpec(memory_space=pltpu.VMEM),
        scratch_shapes=[
            pltpu.VMEM((3 * G_RECV, N), jnp.float32),
            pltpu.SemaphoreType.DMA((18,)),
            pltpu.SemaphoreType.DMA((18,)),
        ],
        compiler_params=pltpu.CompilerParams(collective_id=0),
    )(A, B)


# device time: 32203 ns/iter; 1.0607x vs baseline; 1.0607x over previous
import jax
import jax.numpy as jnp
from jax import lax
from jax.experimental import pallas as pl
from jax.experimental.pallas import tpu as pltpu

N_DEV = 8
M = 768
N = 768

MASKS = (4, 3, 1)
G_ROWS = 256
G_RECV = 256
RECV_OFF = (0, 128, 192)


def _vbit(my, m):
    if m == 4:
        return (my >> 2) & 1
    if m == 3:
        return (my >> 1) & 1
    return (my ^ (my >> 1)) & 1


def kernel(A, B):
    def body(a_ref, b_ref, out_ref, recv_ref, send_sems, recv_sems):
        my = lax.axis_index("i")

        out_ref[:, :] = jnp.dot(
            a_ref[:, :], b_ref[:, :], preferred_element_type=jnp.float32
        )

        barrier = pltpu.get_barrier_semaphore()
        for m in MASKS:
            pl.semaphore_signal(
                barrier, inc=1,
                device_id=(my ^ m,), device_id_type=pl.DeviceIdType.MESH,
            )
        pl.semaphore_wait(barrier, 3)

        stages = (
            ("rs", 0, 128, 0),
            ("rs", 1, 64, 1),
            ("x", 2, 64, 2),
            ("ag", 1, 64, None),
            ("ag", 0, 128, None),
        )

        bases = [jnp.int32(g * G_ROWS) for g in range(3)]
        for s, (kind, ri, size, roff_i) in enumerate(stages):
            started = []
            for g in range(3):
                m = MASKS[(g + ri) % 3]
                bit = _vbit(my, m)
                if kind == "rs":
                    keep = bases[g] + bit * size
                    send = bases[g] + (1 - bit) * size
                    bases[g] = keep
                elif kind == "x":
                    keep = bases[g]
                    send = bases[g]
                else:
                    keep = None
                    send = bases[g]
                    bases[g] = bases[g] - bit * size
                if roff_i is not None:
                    roff = g * G_RECV + RECV_OFF[roff_i]
                    dst = recv_ref.at[pl.ds(roff, size), :]
                else:
                    roff = None
                    dst = out_ref.at[pl.ds(send, size), :]
                rdma = pltpu.make_async_remote_copy(
                    src_ref=out_ref.at[pl.ds(send, size), :],
                    dst_ref=dst,
                    send_sem=send_sems.at[g * 5 + s],
                    recv_sem=recv_sems.at[g * 5 + s],
                    device_id=(my ^ m,),
                    device_id_type=pl.DeviceIdType.MESH,
                )
                rdma.start()
                started.append((rdma, keep, roff))
            for rdma, keep, roff in started:
                rdma.wait()
                if roff is not None:
                    out_ref[pl.ds(keep, size), :] = (
                        out_ref[pl.ds(keep, size), :]
                        + recv_ref[pl.ds(roff, size), :]
                    )

    return pl.pallas_call(
        body,
        out_shape=jax.ShapeDtypeStruct((M, N), jnp.float32),
        in_specs=[
            pl.BlockSpec(memory_space=pltpu.VMEM),
            pl.BlockSpec(memory_space=pltpu.VMEM),
        ],
        out_specs=pl.BlockSpec(memory_space=pltpu.VMEM),
        scratch_shapes=[
            pltpu.VMEM((3 * G_RECV, N), jnp.float32),
            pltpu.SemaphoreType.DMA((15,)),
            pltpu.SemaphoreType.DMA((15,)),
        ],
        compiler_params=pltpu.CompilerParams(collective_id=0),
    )(A, B)


# device time: 25812 ns/iter; 1.3233x vs baseline; 1.2476x over previous
import jax
import jax.numpy as jnp
from jax import lax
from jax.experimental import pallas as pl
from jax.experimental.pallas import tpu as pltpu

N_DEV = 8
M = 768
N = 768

MASKS = (4, 3, 1)
N_GRP = 12
G_ROWS = 64
G_RECV = 64
RECV_OFF = (0, 32, 48)

STAGES = (
    ("rs", 0, 32, 0),
    ("rs", 1, 16, 1),
    ("x", 2, 16, 2),
    ("ag", 1, 16, None),
    ("ag", 0, 32, None),
)


def _vbit(my, m):
    if m == 4:
        return (my >> 2) & 1
    if m == 3:
        return (my >> 1) & 1
    return (my ^ (my >> 1)) & 1


def kernel(A, B):
    def body(a_ref, b_ref, out_ref, recv_ref, send_sems, recv_sems):
        my = lax.axis_index("i")

        barrier = pltpu.get_barrier_semaphore()
        for m in MASKS:
            pl.semaphore_signal(
                barrier, inc=1,
                device_id=(my ^ m,), device_id_type=pl.DeviceIdType.MESH,
            )

        bases = [jnp.int32(g * G_ROWS) for g in range(N_GRP)]

        def start_stage(g, s):
            kind, ri, size, roff_i = STAGES[s]
            m = MASKS[(g + ri) % 3]
            bit = _vbit(my, m)
            if kind == "rs":
                keep = bases[g] + bit * size
                send = bases[g] + (1 - bit) * size
                bases[g] = keep
            elif kind == "x":
                keep = bases[g]
                send = bases[g]
            else:
                keep = None
                send = bases[g]
                bases[g] = bases[g] - bit * size
            if roff_i is not None:
                roff = g * G_RECV + RECV_OFF[roff_i]
                dst = recv_ref.at[pl.ds(roff, size), :]
            else:
                roff = None
                dst = out_ref.at[pl.ds(send, size), :]
            rdma = pltpu.make_async_remote_copy(
                src_ref=out_ref.at[pl.ds(send, size), :],
                dst_ref=dst,
                send_sem=send_sems.at[g * 5 + s],
                recv_sem=recv_sems.at[g * 5 + s],
                device_id=(my ^ m,),
                device_id_type=pl.DeviceIdType.MESH,
            )
            rdma.start()
            return (rdma, keep, roff, size)

        def finish_stage(st):
            rdma, keep, roff, size = st
            rdma.wait()
            if roff is not None:
                out_ref[pl.ds(keep, size), :] = (
                    out_ref[pl.ds(keep, size), :]
                    + recv_ref[pl.ds(roff, size), :]
                )

        WAVES = ((0, 1, 2), (3, 4, 5), (6, 7, 8), (9, 10, 11))
        inflight = {}
        for wi, wave in enumerate(WAVES):
            rows = pl.ds(wi * 3 * G_ROWS, 3 * G_ROWS)
            out_ref[rows, :] = jnp.dot(
                a_ref[rows, :], b_ref[:, :],
                preferred_element_type=jnp.float32,
            )
            if wi == 0:
                pl.semaphore_wait(barrier, 3)
            for g in wave:
                inflight[g] = start_stage(g, 0)
        for s in range(5):
            for wave in WAVES:
                for g in wave:
                    finish_stage(inflight[g])
                    if s < 4:
                        inflight[g] = start_stage(g, s + 1)

    return pl.pallas_call(
        body,
        out_shape=jax.ShapeDtypeStruct((M, N), jnp.float32),
        in_specs=[
            pl.BlockSpec(memory_space=pltpu.VMEM),
            pl.BlockSpec(memory_space=pltpu.VMEM),
        ],
        out_specs=pl.BlockSpec(memory_space=pltpu.VMEM),
        scratch_shapes=[
            pltpu.VMEM((N_GRP * G_RECV, N), jnp.float32),
            pltpu.SemaphoreType.DMA((N_GRP * 5,)),
            pltpu.SemaphoreType.DMA((N_GRP * 5,)),
        ],
        compiler_params=pltpu.CompilerParams(collective_id=0),
    )(A, B)
